# baseline (speedup 1.0000x reference)
import functools

import jax
import jax.numpy as jnp
from jax import lax
from jax.experimental import pallas as pl
from jax.experimental.pallas import tpu as pltpu
from jax.experimental.pallas import tpu_sc as plsc

N = 10000
E = 320000
R = 512
EH = 128
RH = 128

NC = 2
NS = 16
L = 16

CH = 144
B = 32
D = 5
EPT = E // NS
NB = EPT // B


def _prep_body(x_e, x_r, a_h_w, a_t_w, a_r_w, w1, b1, w2, b2,
               av_o, ar_o, m_o, c_o):
    xe = x_e[...]
    xr = x_r[...]
    ah = jnp.sum(xe * a_h_w[0, :][None, :], axis=1)
    at = jnp.sum(xe * a_t_w[0, :][None, :], axis=1)
    ar = jnp.sum(xr * a_r_w[0, :][None, :], axis=1)
    av_o[0, :] = ah
    av_o[1, :] = at
    ar_o[...] = ar

    x1 = lax.dot_general(xr, w1[...], (((1,), (1,)), ((), ())),
                         preferred_element_type=jnp.float32) + b1[...][None, :]
    x2 = lax.dot_general(x1, w2[...], (((1,), (1,)), ((), ())),
                         preferred_element_type=jnp.float32) + b2[...][None, :]
    m = xr + x2
    m_o[:, :RH] = m
    col = lax.broadcasted_iota(jnp.int32, (R, CH - RH), 1)
    m_o[:, RH:] = jnp.where(col == 0, 1.0, 0.0)

    mr = jnp.max(ar)
    ch = jnp.max(ah) + mr
    ct = jnp.max(at) + mr
    ch = jnp.where(ch >= 0.0, ch, 0.01 * ch)
    ct = jnp.where(ct >= 0.0, ct, 0.01 * ct)
    c_o[0, :] = jnp.full((L,), ch, dtype=jnp.float32)
    c_o[1, :] = jnp.full((L,), ct, dtype=jnp.float32)


def _prep(x_e, x_r, a_h_w, a_t_w, a_r_w, w1, b1, w2, b2):
    return pl.pallas_call(
        _prep_body,
        out_shape=(
            jax.ShapeDtypeStruct((2, N), jnp.float32),
            jax.ShapeDtypeStruct((R,), jnp.float32),
            jax.ShapeDtypeStruct((R, CH), jnp.float32),
            jax.ShapeDtypeStruct((2, L), jnp.float32),
        ),
    )(x_e, x_r, a_h_w, a_t_w, a_r_w, w1, b1, w2, b2)


def _edge_body(ei_hbm, rel_hbm, av_hbm, ar_hbm, m_hbm, c_hbm, out_hbm,
               a_v, ar_v, c_v,
               idx0, idx1, idx2, idx3, idx4,
               rel0, rel1, rel2, rel3, rel4,
               msg0, msg1, msg2, msg3, msg4, acc,
               si0, si1, si2, si3, si4,
               sg0, sg1, sg2, sg3, sg4,
               ss0, ss1, ss2, ss3, ss4):
    idx = [idx0, idx1, idx2, idx3, idx4]
    rel = [rel0, rel1, rel2, rel3, rel4]
    msg = [msg0, msg1, msg2, msg3, msg4]
    sem_i = [si0, si1, si2, si3, si4]
    sem_g = [sg0, sg1, sg2, sg3, sg4]
    sem_sc = [ss0, ss1, ss2, ss3, ss4]
    cid = lax.axis_index("c")
    sid = lax.axis_index("s")

    pltpu.sync_copy(av_hbm.at[pl.ds(cid * N, N)], a_v)
    pltpu.sync_copy(ar_hbm, ar_v)
    pltpu.sync_copy(c_hbm, c_v)
    c_vec = c_v[pl.ds(cid * L, L)]

    zero = jnp.zeros((L,), jnp.float32)

    def _zrow(i, _):
        for j in range(CH // L):
            msg0[i, pl.ds(j * L, L)] = zero
        return 0

    lax.fori_loop(0, B, _zrow, 0)

    rows = N // NS
    rbase = sid * rows
    full, tail = rows // B, rows % B
    for k in range(full):
        pltpu.sync_copy(msg0, acc.at[pl.ds(rbase + k * B, B)])
    if tail:
        pltpu.sync_copy(msg0.at[pl.ds(0, tail)],
                        acc.at[pl.ds(rbase + full * B, tail)])
    plsc.subcore_barrier()

    ebase = sid * EPT

    def _load(k, bi, wait):
        a = pltpu.make_async_copy(
            ei_hbm.at[pl.ds(cid * E + ebase + bi * B, B)], idx[k], sem_i[k])
        b = pltpu.make_async_copy(
            rel_hbm.at[pl.ds(ebase + bi * B, B)], rel[k], sem_i[k])
        if wait:
            a.wait()
            b.wait()
        else:
            a.start()
            b.start()

    def _scale(idx_v, rel_v, msg):
        for j in range(B // L):
            eidx = idx_v[pl.ds(j * L, L)]
            ridx = rel_v[pl.ds(j * L, L)]
            g = plsc.load_gather(a_v, [eidx]) + plsc.load_gather(ar_v, [ridx])
            g = jnp.where(g >= 0.0, g, g * 0.01)
            w = jnp.exp(g - c_vec)
            for k in range(L):
                wv = jnp.full((L,), w[k], dtype=jnp.float32)
                row = j * L + k
                for jj in range(CH // L):
                    msg[row, pl.ds(jj * L, L)] = msg[row, pl.ds(jj * L, L)] * wv

    def _gather(k, wait):
        c = pltpu.make_async_copy(m_hbm.at[rel[k]], msg[k], sem_g[k])
        if wait:
            c.wait()
        else:
            c.start()

    def _scatter(k):
        pltpu.async_copy(msg[k], acc.at[idx[k]], sem_sc[k], add=True)

    def _wait_scatter(k):
        pltpu.make_async_copy(msg[k], acc.at[idx[k]], sem_sc[k]).wait()

    def _slot(b0, s, wait_sc, load, gather):
        k3 = (s + 3) % D
        k2 = (s + 2) % D
        if wait_sc:
            _wait_scatter(k3)
        if load:
            _load(k3, b0 + s + 3, False)
        if gather:
            _load(k2, b0 + s + 2, True)
            _gather(k2, False)
        _gather(s, True)
        _scale(idx[s], rel[s], msg[s])
        _scatter(s)

    _load(0, 0, False)
    _load(1, 1, False)
    _load(2, 2, False)
    _load(0, 0, True)
    _gather(0, False)
    _load(1, 1, True)
    _gather(1, False)

    for s in range(D):
        _slot(0, s, s >= 2, True, True)

    def _body(i, _):
        b0 = D * i
        for s in range(D):
            _slot(b0, s, True, True, True)
        return 0

    lax.fori_loop(1, NB // D - 1, _body, 0)

    b0 = NB - D
    for s in range(D):
        _slot(b0, s, True, b0 + s + 3 < NB, b0 + s + 2 < NB)
    _wait_scatter((NB - 2) % D)
    _wait_scatter((NB - 1) % D)

    plsc.subcore_barrier()
    pltpu.sync_copy(acc.at[pl.ds(rbase, rows)],
                    out_hbm.at[cid, pl.ds(rbase, rows)])


_edge_kernel = functools.partial(
    pl.kernel,
    out_type=jax.ShapeDtypeStruct((2, N, CH), jnp.float32),
    mesh=plsc.VectorSubcoreMesh(core_axis_name="c", subcore_axis_name="s",
                                num_cores=NC, num_subcores=NS),
    compiler_params=pltpu.CompilerParams(needs_layout_passes=False,
                                         use_tc_tiling_on_sc=False),
    scratch_types=[
        pltpu.VMEM((N,), jnp.float32),
        pltpu.VMEM((R,), jnp.float32),
        pltpu.VMEM((2 * L,), jnp.float32),
    ] + [pltpu.VMEM((B,), jnp.int32) for _ in range(2 * D)]
      + [pltpu.VMEM((B, CH), jnp.float32) for _ in range(D)]
      + [pltpu.VMEM_SHARED((N, CH), jnp.float32)]
      + [pltpu.SemaphoreType.DMA for _ in range(3 * D)],
)(_edge_body)


def _final_body(scat, r_w, r_b, out):
    h = scat[0]
    t = scat[1]
    xh = h[:, :RH] / (h[:, RH:RH + 1] + 1e-16)
    xt = t[:, :RH] / (t[:, RH:RH + 1] + 1e-16)
    yh = lax.dot_general(xh, r_w[:, :RH], (((1,), (1,)), ((), ())),
                         preferred_element_type=jnp.float32)
    yt = lax.dot_general(xt, r_w[:, RH:], (((1,), (1,)), ((), ())),
                         preferred_element_type=jnp.float32)
    out[...] = yh + yt + r_b[...][None, :]


def _final(scat, r_w, r_b):
    return pl.pallas_call(
        _final_body,
        out_shape=jax.ShapeDtypeStruct((N, 2 * RH), jnp.float32),
    )(scat, r_w, r_b)


def kernel(x_e, x_r, edge_index, rel, rel_all, a_h_w, a_t_w, a_r_w,
           x_r1_w, x_r1_b, x_r2_w, x_r2_b, r_w, r_b):
    del rel_all
    av, ar, m_ext, c = _prep(x_e, x_r, a_h_w, a_t_w, a_r_w,
                             x_r1_w, x_r1_b, x_r2_w, x_r2_b)
    scat = _edge_kernel(edge_index.reshape(-1), rel, av.reshape(-1),
                        ar, m_ext, c.reshape(-1))
    return _final(scat, r_w, r_b)


# speedup vs baseline: 1.1948x; 1.1948x over previous
import functools

import jax
import jax.numpy as jnp
from jax import lax
from jax.experimental import pallas as pl
from jax.experimental.pallas import tpu as pltpu
from jax.experimental.pallas import tpu_sc as plsc

N = 10000
E = 320000
R = 512
EH = 128
RH = 128

NC = 2
NS = 16
L = 16

CH = 144
B = 32
D = 5
EPT = E // NS
NB = EPT // B


def _prep_body(x_e, x_r, a_h_w, a_t_w, a_r_w, w1, b1, w2, b2,
               av_o, ar_o, m_o, c_o):
    xe = x_e[...]
    xr = x_r[...]
    ah = jnp.sum(xe * a_h_w[0, :][None, :], axis=1)
    at = jnp.sum(xe * a_t_w[0, :][None, :], axis=1)
    ar = jnp.sum(xr * a_r_w[0, :][None, :], axis=1)
    av_o[0, :] = ah
    av_o[1, :] = at
    ar_o[...] = ar

    x1 = lax.dot_general(xr, w1[...], (((1,), (1,)), ((), ())),
                         preferred_element_type=jnp.float32) + b1[...][None, :]
    x2 = lax.dot_general(x1, w2[...], (((1,), (1,)), ((), ())),
                         preferred_element_type=jnp.float32) + b2[...][None, :]
    m = xr + x2
    m_o[:, :RH] = m
    col = lax.broadcasted_iota(jnp.int32, (R, CH - RH), 1)
    m_o[:, RH:] = jnp.where(col == 0, 1.0, 0.0)

    mr = jnp.max(ar)
    ch = jnp.max(ah) + mr
    ct = jnp.max(at) + mr
    ch = jnp.where(ch >= 0.0, ch, 0.01 * ch)
    ct = jnp.where(ct >= 0.0, ct, 0.01 * ct)
    c_o[0, :] = jnp.full((L,), ch, dtype=jnp.float32)
    c_o[1, :] = jnp.full((L,), ct, dtype=jnp.float32)


def _prep(x_e, x_r, a_h_w, a_t_w, a_r_w, w1, b1, w2, b2):
    return pl.pallas_call(
        _prep_body,
        out_shape=(
            jax.ShapeDtypeStruct((2, N), jnp.float32),
            jax.ShapeDtypeStruct((R,), jnp.float32),
            jax.ShapeDtypeStruct((R, CH), jnp.float32),
            jax.ShapeDtypeStruct((2, L), jnp.float32),
        ),
    )(x_e, x_r, a_h_w, a_t_w, a_r_w, w1, b1, w2, b2)


def _edge_body(ei_hbm, rel_hbm, av_hbm, ar_hbm, m_hbm, c_hbm, out_hbm,
               a_v, ar_v, c_v,
               idx0, idx1, idx2, idx3, idx4,
               rel0, rel1, rel2, rel3, rel4,
               msg0, msg1, msg2, msg3, msg4, acc,
               si0, si1, si2, si3, si4,
               sg0, sg1, sg2, sg3, sg4,
               ss0, ss1, ss2, ss3, ss4):
    idx = [idx0, idx1, idx2, idx3, idx4]
    rel = [rel0, rel1, rel2, rel3, rel4]
    msg = [msg0, msg1, msg2, msg3, msg4]
    sem_i = [si0, si1, si2, si3, si4]
    sem_g = [sg0, sg1, sg2, sg3, sg4]
    sem_sc = [ss0, ss1, ss2, ss3, ss4]
    cid = lax.axis_index("c")
    sid = lax.axis_index("s")

    pltpu.sync_copy(av_hbm.at[pl.ds(cid * N, N)], a_v)
    pltpu.sync_copy(ar_hbm, ar_v)
    pltpu.sync_copy(c_hbm, c_v)
    c_vec = c_v[pl.ds(cid * L, L)]

    zero = jnp.zeros((L,), jnp.float32)

    def _zrow(i, _):
        for j in range(CH // L):
            msg0[i, pl.ds(j * L, L)] = zero
        return 0

    lax.fori_loop(0, B, _zrow, 0)

    rows = N // NS
    rbase = sid * rows
    full, tail = rows // B, rows % B
    for k in range(full):
        pltpu.sync_copy(msg0, acc.at[pl.ds(rbase + k * B, B)])
    if tail:
        pltpu.sync_copy(msg0.at[pl.ds(0, tail)],
                        acc.at[pl.ds(rbase + full * B, tail)])
    plsc.subcore_barrier()

    ebase = sid * EPT

    def _load(k, bi, wait):
        a = pltpu.make_async_copy(
            ei_hbm.at[pl.ds(cid * E + ebase + bi * B, B)], idx[k], sem_i[k])
        b = pltpu.make_async_copy(
            rel_hbm.at[pl.ds(ebase + bi * B, B)], rel[k], sem_i[k])
        if wait:
            a.wait()
            b.wait()
        else:
            a.start()
            b.start()

    def _scale(idx_v, rel_v, msg):
        def _grp(j, _):
            eidx = idx_v[pl.ds(j * L, L)]
            ridx = rel_v[pl.ds(j * L, L)]
            g = plsc.load_gather(a_v, [eidx]) + plsc.load_gather(ar_v, [ridx])
            g = jnp.where(g >= 0.0, g, g * 0.01)
            w = jnp.exp(g - c_vec)
            for k in range(L):
                wv = jnp.full((L,), w[k], dtype=jnp.float32)
                row = j * L + k
                for jj in range(CH // L):
                    msg[row, pl.ds(jj * L, L)] = msg[row, pl.ds(jj * L, L)] * wv
            return 0

        lax.fori_loop(0, B // L, _grp, 0)

    def _gather(k, wait):
        c = pltpu.make_async_copy(m_hbm.at[rel[k]], msg[k], sem_g[k])
        if wait:
            c.wait()
        else:
            c.start()

    def _scatter(k):
        pltpu.async_copy(msg[k], acc.at[idx[k]], sem_sc[k], add=True)

    def _wait_scatter(k):
        pltpu.make_async_copy(msg[k], acc.at[idx[k]], sem_sc[k]).wait()

    def _slot(b0, s, wait_sc, load, gather):
        k3 = (s + 3) % D
        k2 = (s + 2) % D
        if wait_sc:
            _wait_scatter(k3)
        if load:
            _load(k3, b0 + s + 3, False)
        if gather:
            _load(k2, b0 + s + 2, True)
            _gather(k2, False)
        _gather(s, True)
        _scale(idx[s], rel[s], msg[s])
        _scatter(s)

    _load(0, 0, False)
    _load(1, 1, False)
    _load(2, 2, False)
    _load(0, 0, True)
    _gather(0, False)
    _load(1, 1, True)
    _gather(1, False)

    for s in range(D):
        _slot(0, s, s >= 2, True, True)

    def _body(i, _):
        b0 = D * i
        for s in range(D):
            _slot(b0, s, True, True, True)
        return 0

    lax.fori_loop(1, NB // D - 1, _body, 0)

    b0 = NB - D
    for s in range(D):
        _slot(b0, s, True, b0 + s + 3 < NB, b0 + s + 2 < NB)
    _wait_scatter((NB - 2) % D)
    _wait_scatter((NB - 1) % D)

    plsc.subcore_barrier()
    pltpu.sync_copy(acc.at[pl.ds(rbase, rows)],
                    out_hbm.at[cid, pl.ds(rbase, rows)])


_edge_kernel = functools.partial(
    pl.kernel,
    out_type=jax.ShapeDtypeStruct((2, N, CH), jnp.float32),
    mesh=plsc.VectorSubcoreMesh(core_axis_name="c", subcore_axis_name="s",
                                num_cores=NC, num_subcores=NS),
    compiler_params=pltpu.CompilerParams(needs_layout_passes=False,
                                         use_tc_tiling_on_sc=False),
    scratch_types=[
        pltpu.VMEM((N,), jnp.float32),
        pltpu.VMEM((R,), jnp.float32),
        pltpu.VMEM((2 * L,), jnp.float32),
    ] + [pltpu.VMEM((B,), jnp.int32) for _ in range(2 * D)]
      + [pltpu.VMEM((B, CH), jnp.float32) for _ in range(D)]
      + [pltpu.VMEM_SHARED((N, CH), jnp.float32)]
      + [pltpu.SemaphoreType.DMA for _ in range(3 * D)],
)(_edge_body)


def _final_body(scat, r_w, r_b, out):
    h = scat[0]
    t = scat[1]
    xh = h[:, :RH] / (h[:, RH:RH + 1] + 1e-16)
    xt = t[:, :RH] / (t[:, RH:RH + 1] + 1e-16)
    yh = lax.dot_general(xh, r_w[:, :RH], (((1,), (1,)), ((), ())),
                         preferred_element_type=jnp.float32)
    yt = lax.dot_general(xt, r_w[:, RH:], (((1,), (1,)), ((), ())),
                         preferred_element_type=jnp.float32)
    out[...] = yh + yt + r_b[...][None, :]


def _final(scat, r_w, r_b):
    return pl.pallas_call(
        _final_body,
        out_shape=jax.ShapeDtypeStruct((N, 2 * RH), jnp.float32),
    )(scat, r_w, r_b)


def kernel(x_e, x_r, edge_index, rel, rel_all, a_h_w, a_t_w, a_r_w,
           x_r1_w, x_r1_b, x_r2_w, x_r2_b, r_w, r_b):
    del rel_all
    av, ar, m_ext, c = _prep(x_e, x_r, a_h_w, a_t_w, a_r_w,
                             x_r1_w, x_r1_b, x_r2_w, x_r2_b)
    scat = _edge_kernel(edge_index.reshape(-1), rel, av.reshape(-1),
                        ar, m_ext, c.reshape(-1))
    return _final(scat, r_w, r_b)
